# baseline (speedup 1.0000x reference)
import functools

import jax
import jax.numpy as jnp
from jax.experimental import pallas as pl
from jax.experimental.pallas import tpu as pltpu

_N = 100000
_ROWS = 3125
_PTS_PER_ROW = 32


def _acos(x):
    ax = jnp.abs(x)
    p = jnp.float32(-0.0012624911)
    for c in (0.0066700901, -0.0170881256, 0.0308918810, -0.0501743046,
              0.0889789874, -0.2145988016, 1.5707963050):
        p = p * ax + jnp.float32(c)
    r = jnp.sqrt(jnp.maximum(1.0 - ax, 0.0)) * p
    return jnp.where(x >= 0, r, jnp.float32(jnp.pi) - r)


def _body(t_ref, c_ref, r_ref, p_ref, v_ref, w_ref, b_ref, g_ref, out_ref):
    p = p_ref[...]
    y = jnp.dot(p, w_ref[...], preferred_element_type=jnp.float32)
    y = y + b_ref[...]
    s = jnp.dot(y * y, g_ref[...], preferred_element_type=jnp.float32)
    r = jnp.sqrt(s)
    sum_c = jnp.sum(r[:, :_PTS_PER_ROW])
    sum_r = jnp.sum(r[:, _PTS_PER_ROW:])
    rmse_c = sum_c * (1.0 / _N)
    rmse_r = sum_r * (1.0 / _N)

    gt = t_ref[...]
    deg = 180.0 / jnp.pi

    def scalars(est):
        tr = jnp.sum(est[:3, :3] * gt[:3, :3])
        x = jnp.clip(0.5 * (tr - 1.0), -1.0, 1.0)
        rre = _acos(x) * deg
        dt = gt[:3, 3] - est[:3, 3]
        rte = jnp.sqrt(jnp.sum(dt * dt))
        recall = jnp.logical_and(rre < 15.0, rte < 0.3).astype(jnp.float32)
        return rre, rte, recall

    rre_c, rte_c, recall_c = scalars(c_ref[...])
    rre_r, rte_r, recall_r = scalars(r_ref[...])

    out_ref[0] = rre_c
    out_ref[1] = rte_c
    out_ref[2] = rmse_c
    out_ref[3] = recall_c
    out_ref[4] = rre_r
    out_ref[5] = rte_r
    out_ref[6] = rmse_r
    out_ref[7] = recall_r
    out_ref[8] = v_ref[0]


@jax.jit
def kernel(transform_raw, coarse_trans, refined_trans, src_points, var_rt):
    transform = transform_raw[0]
    pflat = src_points.reshape(_ROWS, 96)

    eye = jnp.eye(_PTS_PER_ROW, dtype=jnp.float32)
    dR_c = coarse_trans[:3, :3] - transform[:3, :3]
    dR_r = refined_trans[:3, :3] - transform[:3, :3]
    W = jnp.concatenate(
        [jnp.kron(eye, dR_c.T), jnp.kron(eye, dR_r.T)], axis=1)
    dt_c = coarse_trans[:3, 3] - transform[:3, 3]
    dt_r = refined_trans[:3, 3] - transform[:3, 3]
    b = jnp.concatenate(
        [jnp.tile(dt_c, _PTS_PER_ROW), jnp.tile(dt_r, _PTS_PER_ROW)]
    ).reshape(1, 192)
    G = jnp.kron(jnp.eye(2 * _PTS_PER_ROW, dtype=jnp.float32),
                 jnp.ones((3, 1), dtype=jnp.float32))

    out = pl.pallas_call(
        _body,
        in_specs=[pl.BlockSpec(memory_space=pltpu.VMEM)] * 4
        + [pl.BlockSpec(memory_space=pltpu.SMEM)]
        + [pl.BlockSpec(memory_space=pltpu.VMEM)] * 3,
        out_specs=pl.BlockSpec(memory_space=pltpu.SMEM),
        out_shape=jax.ShapeDtypeStruct((9,), jnp.float32),
    )(transform, coarse_trans, refined_trans, pflat,
      var_rt, W, b, G)
    return out


# speedup vs baseline: 1.0760x; 1.0760x over previous
import jax
import jax.numpy as jnp
from jax.experimental import pallas as pl
from jax.experimental.pallas import tpu as pltpu

_N = 100000
_ROWS = 3125
_PPR = 32


def _acos(x):
    ax = jnp.abs(x)
    p = jnp.float32(-0.0012624911)
    for c in (0.0066700901, -0.0170881256, 0.0308918810, -0.0501743046,
              0.0889789874, -0.2145988016, 1.5707963050):
        p = p * ax + jnp.float32(c)
    r = jnp.sqrt(jnp.maximum(1.0 - ax, 0.0)) * p
    return jnp.where(x >= 0, r, jnp.float32(jnp.pi) - r)


def _body(t_ref, c_ref, r_ref, v_ref, p_ref, out_ref):
    f32 = jnp.float32

    def delta(est_ref):
        dR = [[est_ref[b, a] - t_ref[b, a] for a in range(3)] for b in range(3)]
        dt = [est_ref[b, 3] - t_ref[b, 3] for b in range(3)]
        return dR, dt

    dR_c, dt_c = delta(c_ref)
    dR_r, dt_r = delta(r_ref)
    dRs = (dR_c, dR_r)
    dts = (dt_c, dt_r)

    i2 = jax.lax.broadcasted_iota(jnp.int32, (96, 192), 0)
    j2 = jax.lax.broadcasted_iota(jnp.int32, (96, 192), 1)
    ci = i2 % 3
    jj = j2 % 96
    rj = jj % 3
    match = (i2 // 3) == (jj // 3)
    half = j2 // 96
    W = jnp.zeros((96, 192), f32)
    for h in range(2):
        for b in range(3):
            for a in range(3):
                cond = match & (ci == a) & (rj == b) & (half == h)
                W = jnp.where(cond, dRs[h][b][a], W)
    jb = jax.lax.broadcasted_iota(jnp.int32, (1, 192), 1)
    rb = (jb % 96) % 3
    hb = jb // 96
    bias = jnp.zeros((1, 192), f32)
    for h in range(2):
        for b in range(3):
            bias = jnp.where((rb == b) & (hb == h), dts[h][b], bias)
    jg = jax.lax.broadcasted_iota(jnp.int32, (192, 64), 0)
    qg = jax.lax.broadcasted_iota(jnp.int32, (192, 64), 1)
    G = ((jg // 3) == qg).astype(f32)

    y = jnp.dot(p_ref[...], W, preferred_element_type=f32) + bias
    s = jnp.dot(y * y, G, preferred_element_type=f32)
    r = jnp.sqrt(s)
    rmse_c = jnp.sum(r[:, :_PPR]) * (1.0 / _N)
    rmse_r = jnp.sum(r[:, _PPR:]) * (1.0 / _N)

    deg = f32(180.0 / jnp.pi)

    def scalars(est_ref):
        tr = f32(0.0)
        for b in range(3):
            for a in range(3):
                tr = tr + est_ref[b, a] * t_ref[b, a]
        x = jnp.clip(0.5 * (tr - 1.0), -1.0, 1.0)
        rre = _acos(x) * deg
        s2 = f32(0.0)
        for b in range(3):
            d = t_ref[b, 3] - est_ref[b, 3]
            s2 = s2 + d * d
        rte = jnp.sqrt(s2)
        recall = jnp.where((rre < 15.0) & (rte < 0.3), f32(1.0), f32(0.0))
        return rre, rte, recall

    rre_c, rte_c, recall_c = scalars(c_ref)
    rre_r, rte_r, recall_r = scalars(r_ref)

    out_ref[0] = rre_c
    out_ref[1] = rte_c
    out_ref[2] = rmse_c
    out_ref[3] = recall_c
    out_ref[4] = rre_r
    out_ref[5] = rte_r
    out_ref[6] = rmse_r
    out_ref[7] = recall_r
    out_ref[8] = v_ref[0]


@jax.jit
def kernel(transform_raw, coarse_trans, refined_trans, src_points, var_rt):
    transform = transform_raw[0]
    pflat = src_points.reshape(_ROWS, 96)

    out = pl.pallas_call(
        _body,
        in_specs=[pl.BlockSpec(memory_space=pltpu.SMEM)] * 4
        + [pl.BlockSpec(memory_space=pltpu.VMEM)],
        out_specs=pl.BlockSpec(memory_space=pltpu.SMEM),
        out_shape=jax.ShapeDtypeStruct((9,), jnp.float32),
    )(transform, coarse_trans, refined_trans, var_rt, pflat)
    return out


# speedup vs baseline: 16.4194x; 15.2590x over previous
import jax
import jax.numpy as jnp
from jax.experimental import pallas as pl
from jax.experimental.pallas import tpu as pltpu

_N = 100000


def _acos(x):
    ax = jnp.abs(x)
    p = jnp.float32(-0.0012624911)
    for c in (0.0066700901, -0.0170881256, 0.0308918810, -0.0501743046,
              0.0889789874, -0.2145988016, 1.5707963050):
        p = p * ax + jnp.float32(c)
    r = jnp.sqrt(jnp.maximum(1.0 - ax, 0.0)) * p
    return jnp.where(x >= 0, r, jnp.float32(jnp.pi) - r)


def _body(t_ref, c_ref, r_ref, v_ref, p_ref, out_ref):
    f32 = jnp.float32

    ji = jax.lax.broadcasted_iota(jnp.int32, (6, 3), 0)
    ai = jax.lax.broadcasted_iota(jnp.int32, (6, 3), 1)
    M = jnp.zeros((6, 3), f32)
    for h, est_ref in enumerate((c_ref, r_ref)):
        for b in range(3):
            for a in range(3):
                M = jnp.where((ji == 3 * h + b) & (ai == a),
                              est_ref[b, a] - t_ref[b, a], M)
    jb = jax.lax.broadcasted_iota(jnp.int32, (6, 1), 0)
    bias = jnp.zeros((6, 1), f32)
    for h, est_ref in enumerate((c_ref, r_ref)):
        for b in range(3):
            bias = jnp.where(jb == 3 * h + b, est_ref[b, 3] - t_ref[b, 3],
                             bias)
    qg = jax.lax.broadcasted_iota(jnp.int32, (2, 6), 0)
    jg = jax.lax.broadcasted_iota(jnp.int32, (2, 6), 1)
    S = ((jg // 3) == qg).astype(f32)

    P = p_ref[...]
    Y = jnp.dot(M, P, preferred_element_type=f32) + bias
    R = jnp.sqrt(jnp.dot(S, Y * Y, preferred_element_type=f32))
    rmse_c = jnp.sum(R[0:1, :]) * (1.0 / _N)
    rmse_r = jnp.sum(R[1:2, :]) * (1.0 / _N)

    deg = f32(180.0 / jnp.pi)

    def scalars(est_ref):
        tr = f32(0.0)
        for b in range(3):
            for a in range(3):
                tr = tr + est_ref[b, a] * t_ref[b, a]
        x = jnp.clip(0.5 * (tr - 1.0), -1.0, 1.0)
        rre = _acos(x) * deg
        s2 = f32(0.0)
        for b in range(3):
            d = t_ref[b, 3] - est_ref[b, 3]
            s2 = s2 + d * d
        rte = jnp.sqrt(s2)
        recall = jnp.where((rre < 15.0) & (rte < 0.3), f32(1.0), f32(0.0))
        return rre, rte, recall

    rre_c, rte_c, recall_c = scalars(c_ref)
    rre_r, rte_r, recall_r = scalars(r_ref)

    out_ref[0] = rre_c
    out_ref[1] = rte_c
    out_ref[2] = rmse_c
    out_ref[3] = recall_c
    out_ref[4] = rre_r
    out_ref[5] = rte_r
    out_ref[6] = rmse_r
    out_ref[7] = recall_r
    out_ref[8] = v_ref[0]


@jax.jit
def kernel(transform_raw, coarse_trans, refined_trans, src_points, var_rt):
    transform = transform_raw[0]
    planes = src_points.T

    out = pl.pallas_call(
        _body,
        in_specs=[pl.BlockSpec(memory_space=pltpu.SMEM)] * 4
        + [pl.BlockSpec(memory_space=pltpu.VMEM)],
        out_specs=pl.BlockSpec(memory_space=pltpu.SMEM),
        out_shape=jax.ShapeDtypeStruct((9,), jnp.float32),
    )(transform, coarse_trans, refined_trans, var_rt, planes)
    return out
